# speedup vs baseline: 6.3223x; 6.3223x over previous
import functools

import jax
import jax.numpy as jnp
from jax import lax
from jax.experimental import pallas as pl
from jax.experimental.pallas import tpu as pltpu
from jax.experimental.pallas import tpu_sc as plsc

NUM_EMBEDDINGS = 100000
EMBED_DIM = 128
BATCH = 4096
HIST = 200

NC = 2
NS = 16
NW = NC * NS

CHUNK = 128
N_FLAT = BATCH * HIST
ROWS_PER_W = N_FLAT // NW
CHUNKS_PER_W = ROWS_PER_W // CHUNK


def _gather_body(x_hbm, table_hbm, out_hbm, idx_v, rows_v, gsem):
    wid = lax.axis_index("s") * NC + lax.axis_index("c")
    base_chunk = wid * CHUNKS_PER_W
    pltpu.sync_copy(x_hbm.at[pl.ds(base_chunk, CHUNKS_PER_W)], idx_v)

    def body(j, carry):
        pltpu.async_copy(table_hbm.at[idx_v.at[j]], rows_v, gsem).wait()
        pltpu.sync_copy(
            rows_v, out_hbm.at[pl.ds((base_chunk + j) * CHUNK, CHUNK)]
        )
        return carry

    lax.fori_loop(0, CHUNKS_PER_W, body, 0, unroll=False)


@jax.jit
def _embed_lookup(x2d, table):
    mesh = plsc.VectorSubcoreMesh(
        core_axis_name="c", subcore_axis_name="s", num_cores=NC, num_subcores=NS
    )
    run = pl.kernel(
        _gather_body,
        out_type=jax.ShapeDtypeStruct((N_FLAT, EMBED_DIM), jnp.float32),
        mesh=mesh,
        scratch_types=[
            pltpu.VMEM((CHUNKS_PER_W, CHUNK), jnp.int32),
            pltpu.VMEM((CHUNK, EMBED_DIM), jnp.float32),
            pltpu.SemaphoreType.DMA,
        ],
    )
    return run(x2d, table)


def kernel(x, table):
    x2d = x.reshape(N_FLAT // CHUNK, CHUNK)
    out = _embed_lookup(x2d, table)
    return out.reshape(BATCH, HIST, EMBED_DIM)


# speedup vs baseline: 9.2837x; 1.4684x over previous
import functools

import jax
import jax.numpy as jnp
from jax import lax
from jax.experimental import pallas as pl
from jax.experimental.pallas import tpu as pltpu
from jax.experimental.pallas import tpu_sc as plsc

NUM_EMBEDDINGS = 100000
EMBED_DIM = 128
BATCH = 4096
HIST = 200

NC = 2
NS = 16
NW = NC * NS

CHUNK = 128
N_FLAT = BATCH * HIST
ROWS_PER_W = N_FLAT // NW
CHUNKS_PER_W = ROWS_PER_W // CHUNK
NBUF = 4
NGROUPS = CHUNKS_PER_W // NBUF


def _gather_body(x_hbm, table_hbm, out_hbm, idx_v, rows_v, gsems):
    wid = lax.axis_index("s") * NC + lax.axis_index("c")
    base_chunk = wid * CHUNKS_PER_W
    pltpu.sync_copy(x_hbm.at[pl.ds(base_chunk, CHUNKS_PER_W)], idx_v)

    def fire(j, b):
        pltpu.async_copy(table_hbm.at[idx_v.at[j]], rows_v.at[b], gsems.at[b])

    def drain_and_store(j, b):
        pltpu.make_async_copy(
            table_hbm.at[idx_v.at[j]], rows_v.at[b], gsems.at[b]
        ).wait()
        pltpu.sync_copy(
            rows_v.at[b], out_hbm.at[pl.ds((base_chunk + j) * CHUNK, CHUNK)]
        )

    for b in range(NBUF):
        fire(b, b)

    def group(g, carry):
        for b in range(NBUF):
            j = g * NBUF + b
            drain_and_store(j, b)
            fire(j + NBUF, b)
        return carry

    lax.fori_loop(0, NGROUPS - 1, group, 0, unroll=False)

    for b in range(NBUF):
        drain_and_store((NGROUPS - 1) * NBUF + b, b)


@jax.jit
def _embed_lookup(x2d, table):
    mesh = plsc.VectorSubcoreMesh(
        core_axis_name="c", subcore_axis_name="s", num_cores=NC, num_subcores=NS
    )
    run = pl.kernel(
        _gather_body,
        out_type=jax.ShapeDtypeStruct((N_FLAT, EMBED_DIM), jnp.float32),
        mesh=mesh,
        scratch_types=[
            pltpu.VMEM((CHUNKS_PER_W, CHUNK), jnp.int32),
            pltpu.VMEM((NBUF, CHUNK, EMBED_DIM), jnp.float32),
            pltpu.SemaphoreType.DMA((NBUF,)),
        ],
    )
    return run(x2d, table)


def kernel(x, table):
    x2d = x.reshape(N_FLAT // CHUNK, CHUNK)
    out = _embed_lookup(x2d, table)
    return out.reshape(BATCH, HIST, EMBED_DIM)
